# baseline (speedup 1.0000x reference)
import jax
import jax.numpy as jnp
from jax import lax
from jax.experimental import pallas as pl
from jax.experimental.pallas import tpu as pltpu
from jax.experimental.pallas import tpu_sc as plsc

N = 10000
E = 320000
D = 128
S = 2048
DG = 16
DE = D + DG

NC = 2
NS = 16
NW = NC * NS
EW = E // NW
K = 80
CPS = 25
NSUPER = EW // (K * CPS)
ROWS_T = N // NS
SW = S // NW
LANES = 16


def _encode_body(src_h, dst_h, typ_h, nodep_h, relnp_h,
                 agg0_h, agg1_h,
                 agg_sh, sidx, didx, tidx,
                 br0, br1, br2,
                 semb0, semb1, semb2, semc0, semc1, semc2,
                 semd0, semd1, semd2):
    c = lax.axis_index("c")
    s = lax.axis_index("s")
    wid = s * NC + c
    br = [br0, br1, br2]
    semb = [semb0, semb1, semb2]
    semc = [semc0, semc1, semc2]
    semd = [semd0, semd1, semd2]
    zero = jnp.zeros((LANES,), jnp.float32)
    zeroi = jnp.zeros((LANES,), jnp.int32)

    def zb_body(j, carry):
        for i in range(DE // LANES):
            br0[j, pl.ds(i * LANES, LANES)] = zero
        return carry

    lax.fori_loop(0, K, zb_body, None)

    def didx_body(j, carry):
        for i in range(K // LANES):
            didx[j, pl.ds(i * LANES, LANES)] = zeroi
        return carry

    lax.fori_loop(0, CPS, didx_body, None)

    for q in range(7):
        pltpu.sync_copy(br0, agg_sh.at[pl.ds(s * ROWS_T + q * K, K)])
    pltpu.sync_copy(br0.at[pl.ds(0, 65)],
                    agg_sh.at[pl.ds(s * ROWS_T + 7 * K, 65)])
    plsc.subcore_barrier()

    for p in (2, 0):
        pltpu.async_copy(br0, agg_sh.at[didx.at[0]], semd[p], add=True)

    def block(g, carry):
        for p in (2, 0):
            pltpu.make_async_copy(br0, agg_sh.at[didx.at[0]], semd[p]).wait()
        rbase = wid * (EW // K) + g * CPS
        pltpu.sync_copy(src_h.at[pl.ds(rbase, CPS)], sidx)
        pltpu.sync_copy(typ_h.at[pl.ds(rbase, CPS)], tidx)
        pltpu.sync_copy(dst_h.at[pl.ds(rbase, CPS)], didx)
        cpn = [None, None, None]
        cpn[0] = pltpu.async_copy(nodep_h.at[sidx.at[0]], br[0], semb[0])
        for q in range(CPS):
            p = q % 3
            if q >= 2:
                pp = (q - 2) % 3
                pltpu.make_async_copy(
                    br0, agg_sh.at[didx.at[0]], semd[pp]).wait()
            if q + 1 < CPS:
                pn = (q + 1) % 3
                cpn[pn] = pltpu.async_copy(nodep_h.at[sidx.at[q + 1]],
                                           br[pn], semb[pn])
            cpn[p].wait()
            cpc = pltpu.async_copy(relnp_h.at[tidx.at[q]], br[p],
                                   semc[p], add=True)
            cpc.wait()
            pltpu.async_copy(br[p], agg_sh.at[didx.at[q]], semd[p], add=True)
        return carry

    lax.fori_loop(0, NSUPER, block, None)
    for p in (2, 0):
        pltpu.make_async_copy(br0, agg_sh.at[didx.at[0]], semd[p]).wait()
    plsc.subcore_barrier()

    @pl.when(c == 0)
    def _():
        pltpu.sync_copy(agg_sh.at[pl.ds(s * ROWS_T, ROWS_T)],
                        agg0_h.at[pl.ds(s * ROWS_T, ROWS_T)])

    @pl.when(c == 1)
    def _():
        pltpu.sync_copy(agg_sh.at[pl.ds(s * ROWS_T, ROWS_T)],
                        agg1_h.at[pl.ds(s * ROWS_T, ROWS_T)])


def _extract_body(sub_h, a0_h, a1_h, node_h, x_h,
                  idx, g0, g1, gn, xb, sem):
    c = lax.axis_index("c")
    s = lax.axis_index("s")
    wid = s * NC + c
    base = wid * SW
    pltpu.sync_copy(sub_h.at[pl.ds(base, SW)], idx)
    cps = [pltpu.async_copy(a0_h.at[idx], g0, sem),
           pltpu.async_copy(a1_h.at[idx], g1, sem),
           pltpu.async_copy(node_h.at[idx], gn, sem)]
    for cp in cps:
        cp.wait()

    def row(j, carry):
        dsl = pl.ds(D, LANES)
        deg = g0[j, dsl] + g1[j, dsl]
        rcp = 1.0 / jnp.maximum(deg, 1.0)
        for i in range(D // LANES):
            sl = pl.ds(i * LANES, LANES)
            xb[j, sl] = (g0[j, sl] + g1[j, sl]) * rcp + gn[j, sl]
        return carry

    lax.fori_loop(0, SW, row, None)
    pltpu.sync_copy(xb, x_h.at[pl.ds(base, SW)])


def _matmul_body(x_ref, w_ref, o_ref):
    o_ref[...] = jnp.tanh(
        jnp.dot(x_ref[...], w_ref[...], preferred_element_type=jnp.float32))


@jax.jit
def kernel(edge_index, edge_type, subgraph_nodes, node_emb, rel_emb, W):
    src = edge_index[0].reshape(E // K, K)
    dst = edge_index[1].reshape(E // K, K)
    etype = edge_type.reshape(E // K, K)
    node_pad = jnp.concatenate(
        [node_emb, jnp.ones((N, DG), jnp.float32)], axis=1)
    reln_pad = jnp.concatenate(
        [-rel_emb, jnp.zeros((rel_emb.shape[0], DG), jnp.float32)], axis=1)

    mesh = plsc.VectorSubcoreMesh(core_axis_name="c", subcore_axis_name="s")
    encode = pl.kernel(
        _encode_body,
        out_type=[jax.ShapeDtypeStruct((N, DE), jnp.float32),
                  jax.ShapeDtypeStruct((N, DE), jnp.float32)],
        mesh=mesh,
        scratch_types=[
            pltpu.VMEM_SHARED((N, DE), jnp.float32),
            pltpu.VMEM((CPS, K), jnp.int32),
            pltpu.VMEM((CPS, K), jnp.int32),
            pltpu.VMEM((CPS, K), jnp.int32),
            pltpu.VMEM((K, DE), jnp.float32),
            pltpu.VMEM((K, DE), jnp.float32),
            pltpu.VMEM((K, DE), jnp.float32),
        ] + [pltpu.SemaphoreType.DMA] * 9,
        compiler_params=pltpu.CompilerParams(use_tc_tiling_on_sc=False),
    )
    agg0, agg1 = encode(src, dst, etype, node_pad, reln_pad)

    extract = pl.kernel(
        _extract_body,
        out_type=jax.ShapeDtypeStruct((S, D), jnp.float32),
        mesh=plsc.VectorSubcoreMesh(core_axis_name="c", subcore_axis_name="s"),
        scratch_types=[
            pltpu.VMEM((SW,), jnp.int32),
            pltpu.VMEM((SW, DE), jnp.float32),
            pltpu.VMEM((SW, DE), jnp.float32),
            pltpu.VMEM((SW, D), jnp.float32),
            pltpu.VMEM((SW, D), jnp.float32),
            pltpu.SemaphoreType.DMA,
        ],
        compiler_params=pltpu.CompilerParams(use_tc_tiling_on_sc=False),
    )
    x = extract(subgraph_nodes, agg0, agg1, node_emb)

    return pl.pallas_call(
        _matmul_body,
        out_shape=jax.ShapeDtypeStruct((S, D), jnp.float32),
    )(x, W)


# speedup vs baseline: 1.3080x; 1.3080x over previous
import jax
import jax.numpy as jnp
from jax import lax
from jax.experimental import pallas as pl
from jax.experimental.pallas import tpu as pltpu
from jax.experimental.pallas import tpu_sc as plsc

N = 10000
E = 320000
D = 128
S = 2048
DG = 16

NC = 2
NS = 16
NW = NC * NS
EW = E // NW
K = 80
CPS = 25
NSUPER = EW // (K * CPS)
ROWS_T = N // NS
SW = S // NW
LANES = 16


def _encode_body(src_h, dstf_h, typ_h, node_h, reln_h,
                 agg0_h, agg1_h, deg0_h, deg1_h,
                 agg_sh, deg_sh, rel_sh, sidx, didxf, tidx,
                 br0, br1, ones, zdeg,
                 semb0, semb1, semc0, semc1, semd0, semd1, seme0):
    c = lax.axis_index("c")
    s = lax.axis_index("s")
    wid = s * NC + c
    br = [br0, br1]
    semb = [semb0, semb1]
    semc = [semc0, semc1]
    semd = [semd0, semd1]
    zero = jnp.zeros((LANES,), jnp.float32)
    zeroi = jnp.zeros((LANES,), jnp.int32)
    one = jnp.ones((LANES,), jnp.float32)

    def zagg_body(j, carry):
        for i in range(D // LANES):
            br0[j, pl.ds(i * LANES, LANES)] = zero
        return carry

    lax.fori_loop(0, K, zagg_body, None)

    def zdeg_body(j, carry):
        zdeg[j, pl.ds(0, LANES)] = zero
        return carry

    lax.fori_loop(0, K, zdeg_body, None)

    def ones_body(j, carry):
        ones[j, pl.ds(0, LANES)] = one
        return carry

    lax.fori_loop(0, 8 * K, ones_body, None)

    def didx_body(j, carry):
        didxf[pl.ds(j * LANES, LANES)] = zeroi
        return carry

    lax.fori_loop(0, CPS * K // LANES, didx_body, None)

    for q in range(7):
        pltpu.sync_copy(br0, agg_sh.at[pl.ds(s * ROWS_T + q * K, K)])
        pltpu.sync_copy(zdeg, deg_sh.at[pl.ds(s * ROWS_T + q * K, K)])
    pltpu.sync_copy(br0.at[pl.ds(0, 65)],
                    agg_sh.at[pl.ds(s * ROWS_T + 7 * K, 65)])
    pltpu.sync_copy(zdeg.at[pl.ds(0, 65)],
                    deg_sh.at[pl.ds(s * ROWS_T + 7 * K, 65)])

    @pl.when(s == 0)
    def _():
        pltpu.sync_copy(reln_h, rel_sh)
    plsc.subcore_barrier()

    for p in range(2):
        pltpu.async_copy(br0, agg_sh.at[didxf.at[pl.ds(0, K)]], semd[p], add=True)

    def block(g, carry):
        for p in range(2):
            pltpu.make_async_copy(br0, agg_sh.at[didxf.at[pl.ds(0, K)]], semd[p]).wait()

        @pl.when(g > 0)
        def _():
            for _i in range(3):
                pltpu.make_async_copy(
                    ones, deg_sh.at[didxf.at[pl.ds(0, 8 * K)]], seme0).wait()
            pltpu.make_async_copy(
                ones.at[pl.ds(0, K)],
                deg_sh.at[didxf.at[pl.ds(0, K)]], seme0).wait()

        rbase = wid * (EW // K) + g * CPS
        pltpu.sync_copy(src_h.at[pl.ds(rbase, CPS)], sidx)
        pltpu.sync_copy(typ_h.at[pl.ds(rbase, CPS)], tidx)
        pltpu.sync_copy(dstf_h.at[pl.ds(wid * EW + g * CPS * K, CPS * K)],
                        didxf)

        for b in range(3):
            pltpu.async_copy(ones, deg_sh.at[didxf.at[pl.ds(8 * K * b, 8 * K)]],
                             seme0, add=True)
        pltpu.async_copy(ones.at[pl.ds(0, K)],
                         deg_sh.at[didxf.at[pl.ds(24 * K, K)]],
                         seme0, add=True)
        pend_c = None
        for q in range(CPS):
            p = q & 1
            if q >= 2:
                pltpu.make_async_copy(
                    br0, agg_sh.at[didxf.at[pl.ds(0, K)]], semd[p]).wait()
            cpb = pltpu.async_copy(rel_sh.at[tidx.at[q]], br[p], semb[p])
            if pend_c is not None:
                qq, pp, cpc = pend_c
                cpc.wait()
                pltpu.async_copy(br[pp],
                                 agg_sh.at[didxf.at[pl.ds(qq * K, K)]],
                                 semd[pp], add=True)
            cpb.wait()
            cpc = pltpu.async_copy(node_h.at[sidx.at[q]], br[p],
                                   semc[p], add=True)
            pend_c = (q, p, cpc)
        qq, pp, cpc = pend_c
        cpc.wait()
        pltpu.async_copy(br[pp], agg_sh.at[didxf.at[pl.ds(qq * K, K)]], semd[pp], add=True)
        return carry

    lax.fori_loop(0, NSUPER, block, None)
    for p in range(2):
        pltpu.make_async_copy(br0, agg_sh.at[didxf.at[pl.ds(0, K)]], semd[p]).wait()
    for _i in range(3):
        pltpu.make_async_copy(
            ones, deg_sh.at[didxf.at[pl.ds(0, 8 * K)]], seme0).wait()
    pltpu.make_async_copy(
        ones.at[pl.ds(0, K)],
        deg_sh.at[didxf.at[pl.ds(0, K)]], seme0).wait()
    plsc.subcore_barrier()

    @pl.when(c == 0)
    def _():
        pltpu.sync_copy(agg_sh.at[pl.ds(s * ROWS_T, ROWS_T)],
                        agg0_h.at[pl.ds(s * ROWS_T, ROWS_T)])
        pltpu.sync_copy(deg_sh.at[pl.ds(s * ROWS_T, ROWS_T)],
                        deg0_h.at[pl.ds(s * ROWS_T, ROWS_T)])

    @pl.when(c == 1)
    def _():
        pltpu.sync_copy(agg_sh.at[pl.ds(s * ROWS_T, ROWS_T)],
                        agg1_h.at[pl.ds(s * ROWS_T, ROWS_T)])
        pltpu.sync_copy(deg_sh.at[pl.ds(s * ROWS_T, ROWS_T)],
                        deg1_h.at[pl.ds(s * ROWS_T, ROWS_T)])


def _extract_body(sub_h, a0_h, a1_h, d0_h, d1_h, node_h, x_h,
                  idx, g0, g1, d0, d1, gn, xb, sem):
    c = lax.axis_index("c")
    s = lax.axis_index("s")
    wid = s * NC + c
    base = wid * SW
    pltpu.sync_copy(sub_h.at[pl.ds(base, SW)], idx)
    cps = [pltpu.async_copy(a0_h.at[idx], g0, sem),
           pltpu.async_copy(a1_h.at[idx], g1, sem),
           pltpu.async_copy(d0_h.at[idx], d0, sem),
           pltpu.async_copy(d1_h.at[idx], d1, sem),
           pltpu.async_copy(node_h.at[idx], gn, sem)]
    for cp in cps:
        cp.wait()

    def row(j, carry):
        deg = d0[j, pl.ds(0, LANES)] + d1[j, pl.ds(0, LANES)]
        rcp = 1.0 / jnp.maximum(deg, 1.0)
        for i in range(D // LANES):
            sl = pl.ds(i * LANES, LANES)
            xb[j, sl] = (g0[j, sl] + g1[j, sl]) * rcp + gn[j, sl]
        return carry

    lax.fori_loop(0, SW, row, None)
    pltpu.sync_copy(xb, x_h.at[pl.ds(base, SW)])


def _matmul_body(x_ref, w_ref, o_ref):
    o_ref[...] = jnp.tanh(
        jnp.dot(x_ref[...], w_ref[...], preferred_element_type=jnp.float32))


@jax.jit
def kernel(edge_index, edge_type, subgraph_nodes, node_emb, rel_emb, W):
    src = edge_index[0].reshape(E // K, K)
    dstf = edge_index[1]
    etype = edge_type.reshape(E // K, K)
    rel_neg = -rel_emb

    mesh = plsc.VectorSubcoreMesh(core_axis_name="c", subcore_axis_name="s")
    encode = pl.kernel(
        _encode_body,
        out_type=[jax.ShapeDtypeStruct((N, D), jnp.float32),
                  jax.ShapeDtypeStruct((N, D), jnp.float32),
                  jax.ShapeDtypeStruct((N, DG), jnp.float32),
                  jax.ShapeDtypeStruct((N, DG), jnp.float32)],
        mesh=mesh,
        scratch_types=[
            pltpu.VMEM_SHARED((N, D), jnp.float32),
            pltpu.VMEM_SHARED((N, DG), jnp.float32),
            pltpu.VMEM_SHARED((200, D), jnp.float32),
            pltpu.VMEM((CPS, K), jnp.int32),
            pltpu.VMEM((CPS * K,), jnp.int32),
            pltpu.VMEM((CPS, K), jnp.int32),
            pltpu.VMEM((K, D), jnp.float32),
            pltpu.VMEM((K, D), jnp.float32),
            pltpu.VMEM((8 * K, DG), jnp.float32),
            pltpu.VMEM((K, DG), jnp.float32),
        ] + [pltpu.SemaphoreType.DMA] * 7,
        compiler_params=pltpu.CompilerParams(use_tc_tiling_on_sc=False),
    )
    agg0, agg1, deg0, deg1 = encode(src, dstf, etype, node_emb, rel_neg)

    extract = pl.kernel(
        _extract_body,
        out_type=jax.ShapeDtypeStruct((S, D), jnp.float32),
        mesh=plsc.VectorSubcoreMesh(core_axis_name="c", subcore_axis_name="s"),
        scratch_types=[
            pltpu.VMEM((SW,), jnp.int32),
            pltpu.VMEM((SW, D), jnp.float32),
            pltpu.VMEM((SW, D), jnp.float32),
            pltpu.VMEM((SW, DG), jnp.float32),
            pltpu.VMEM((SW, DG), jnp.float32),
            pltpu.VMEM((SW, D), jnp.float32),
            pltpu.VMEM((SW, D), jnp.float32),
            pltpu.SemaphoreType.DMA,
        ],
        compiler_params=pltpu.CompilerParams(use_tc_tiling_on_sc=False),
    )
    x = extract(subgraph_nodes, agg0, agg1, deg0, deg1, node_emb)

    return pl.pallas_call(
        _matmul_body,
        out_shape=jax.ShapeDtypeStruct((S, D), jnp.float32),
    )(x, W)


# speedup vs baseline: 1.3683x; 1.0461x over previous
import jax
import jax.numpy as jnp
from jax import lax
from jax.experimental import pallas as pl
from jax.experimental.pallas import tpu as pltpu
from jax.experimental.pallas import tpu_sc as plsc

N = 10000
E = 320000
D = 128
S = 2048
DG = 8
DE = D + DG

NC = 2
NS = 16
NW = NC * NS
EW = E // NW
K = 80
CPS = 25
NSUPER = EW // (K * CPS)
ROWS_T = N // NS
SW = S // NW
LANES = 16

ROW_OFFS = tuple(range(0, D, LANES)) + (DE - LANES,)


def _encode_body(src_h, dst_h, typ_h, nodep_h, relnp_h,
                 agg0_h, agg1_h,
                 agg_sh, rel_sh, sidx, didx, tidx,
                 br0, br1, br2,
                 semb0, semb1, semb2, semc0, semc1, semc2,
                 semd0, semd1, semd2):
    c = lax.axis_index("c")
    s = lax.axis_index("s")
    wid = s * NC + c
    br = [br0, br1, br2]
    semb = [semb0, semb1, semb2]
    semc = [semc0, semc1, semc2]
    semd = [semd0, semd1, semd2]
    zero = jnp.zeros((LANES,), jnp.float32)
    zeroi = jnp.zeros((LANES,), jnp.int32)

    def zb_body(j, carry):
        for off in ROW_OFFS:
            br0[j, pl.ds(off, LANES)] = zero
        return carry

    lax.fori_loop(0, K, zb_body, None)

    def didx_body(j, carry):
        for i in range(K // LANES):
            didx[j, pl.ds(i * LANES, LANES)] = zeroi
        return carry

    lax.fori_loop(0, CPS, didx_body, None)

    for q in range(7):
        pltpu.sync_copy(br0, agg_sh.at[pl.ds(s * ROWS_T + q * K, K)])
    pltpu.sync_copy(br0.at[pl.ds(0, 65)],
                    agg_sh.at[pl.ds(s * ROWS_T + 7 * K, 65)])

    @pl.when(s == 0)
    def _():
        pltpu.sync_copy(relnp_h, rel_sh)
    plsc.subcore_barrier()

    for p in (2, 0):
        pltpu.async_copy(br0, agg_sh.at[didx.at[0]], semd[p], add=True)

    def block(g, carry):
        for p in (2, 0):
            pltpu.make_async_copy(br0, agg_sh.at[didx.at[0]], semd[p]).wait()
        rbase = wid * (EW // K) + g * CPS
        pltpu.sync_copy(src_h.at[pl.ds(rbase, CPS)], sidx)
        pltpu.sync_copy(typ_h.at[pl.ds(rbase, CPS)], tidx)
        pltpu.sync_copy(dst_h.at[pl.ds(rbase, CPS)], didx)
        cpn = [None, None, None]
        cpn[0] = pltpu.async_copy(nodep_h.at[sidx.at[0]], br[0], semb[0])
        for q in range(CPS):
            p = q % 3
            if q >= 2:
                pp = (q - 2) % 3
                pltpu.make_async_copy(
                    br0, agg_sh.at[didx.at[0]], semd[pp]).wait()
            if q + 1 < CPS:
                pn = (q + 1) % 3
                cpn[pn] = pltpu.async_copy(nodep_h.at[sidx.at[q + 1]],
                                           br[pn], semb[pn])
            cpn[p].wait()
            cpc = pltpu.async_copy(rel_sh.at[tidx.at[q]], br[p],
                                   semc[p], add=True)
            cpc.wait()
            pltpu.async_copy(br[p], agg_sh.at[didx.at[q]], semd[p], add=True)
        return carry

    lax.fori_loop(0, NSUPER, block, None)
    for p in (2, 0):
        pltpu.make_async_copy(br0, agg_sh.at[didx.at[0]], semd[p]).wait()
    plsc.subcore_barrier()

    @pl.when(c == 0)
    def _():
        pltpu.sync_copy(agg_sh.at[pl.ds(s * ROWS_T, ROWS_T)],
                        agg0_h.at[pl.ds(s * ROWS_T, ROWS_T)])

    @pl.when(c == 1)
    def _():
        pltpu.sync_copy(agg_sh.at[pl.ds(s * ROWS_T, ROWS_T)],
                        agg1_h.at[pl.ds(s * ROWS_T, ROWS_T)])


def _extract_body(sub_h, a0_h, a1_h, node_h, x_h,
                  idx, g0, g1, gn, xb, sem):
    c = lax.axis_index("c")
    s = lax.axis_index("s")
    wid = s * NC + c
    base = wid * SW
    pltpu.sync_copy(sub_h.at[pl.ds(base, SW)], idx)
    cps = [pltpu.async_copy(a0_h.at[idx], g0, sem),
           pltpu.async_copy(a1_h.at[idx], g1, sem),
           pltpu.async_copy(node_h.at[idx], gn, sem)]
    for cp in cps:
        cp.wait()

    def row(j, carry):
        dpos = jnp.full((LANES,), D, jnp.int32)
        jv = jnp.full((LANES,), j, jnp.int32)
        deg = (plsc.load_gather(g0, [jv, dpos]) +
               plsc.load_gather(g1, [jv, dpos]))
        rcp = 1.0 / jnp.maximum(deg, 1.0)
        for i in range(D // LANES):
            sl = pl.ds(i * LANES, LANES)
            xb[j, sl] = (g0[j, sl] + g1[j, sl]) * rcp + gn[j, sl]
        return carry

    lax.fori_loop(0, SW, row, None)
    pltpu.sync_copy(xb, x_h.at[pl.ds(base, SW)])


def _matmul_body(x_ref, w_ref, o_ref):
    o_ref[...] = jnp.tanh(
        jnp.dot(x_ref[...], w_ref[...], preferred_element_type=jnp.float32))


@jax.jit
def kernel(edge_index, edge_type, subgraph_nodes, node_emb, rel_emb, W):
    src = edge_index[0].reshape(E // K, K)
    dst = edge_index[1].reshape(E // K, K)
    etype = edge_type.reshape(E // K, K)
    node_pad = jnp.concatenate(
        [node_emb, jnp.ones((N, DG), jnp.float32)], axis=1)
    reln_pad = jnp.concatenate(
        [-rel_emb, jnp.zeros((rel_emb.shape[0], DG), jnp.float32)], axis=1)

    mesh = plsc.VectorSubcoreMesh(core_axis_name="c", subcore_axis_name="s")
    encode = pl.kernel(
        _encode_body,
        out_type=[jax.ShapeDtypeStruct((N, DE), jnp.float32),
                  jax.ShapeDtypeStruct((N, DE), jnp.float32)],
        mesh=mesh,
        scratch_types=[
            pltpu.VMEM_SHARED((N, DE), jnp.float32),
            pltpu.VMEM_SHARED((200, DE), jnp.float32),
            pltpu.VMEM((CPS, K), jnp.int32),
            pltpu.VMEM((CPS, K), jnp.int32),
            pltpu.VMEM((CPS, K), jnp.int32),
            pltpu.VMEM((K, DE), jnp.float32),
            pltpu.VMEM((K, DE), jnp.float32),
            pltpu.VMEM((K, DE), jnp.float32),
        ] + [pltpu.SemaphoreType.DMA] * 9,
        compiler_params=pltpu.CompilerParams(use_tc_tiling_on_sc=False),
    )
    agg0, agg1 = encode(src, dst, etype, node_pad, reln_pad)

    extract = pl.kernel(
        _extract_body,
        out_type=jax.ShapeDtypeStruct((S, D), jnp.float32),
        mesh=plsc.VectorSubcoreMesh(core_axis_name="c", subcore_axis_name="s"),
        scratch_types=[
            pltpu.VMEM((SW,), jnp.int32),
            pltpu.VMEM((SW, DE), jnp.float32),
            pltpu.VMEM((SW, DE), jnp.float32),
            pltpu.VMEM((SW, D), jnp.float32),
            pltpu.VMEM((SW, D), jnp.float32),
            pltpu.SemaphoreType.DMA,
        ],
        compiler_params=pltpu.CompilerParams(use_tc_tiling_on_sc=False,
                                             needs_layout_passes=False),
    )
    x = extract(subgraph_nodes, agg0, agg1, node_emb)

    return pl.pallas_call(
        _matmul_body,
        out_shape=jax.ShapeDtypeStruct((S, D), jnp.float32),
    )(x, W)


# speedup vs baseline: 1.3798x; 1.0084x over previous
import jax
import jax.numpy as jnp
from jax import lax
from jax.experimental import pallas as pl
from jax.experimental.pallas import tpu as pltpu
from jax.experimental.pallas import tpu_sc as plsc

N = 10000
E = 320000
D = 128
S = 2048
DG = 8
DE = D + DG

NC = 2
NS = 16
NW = NC * NS
EW = E // NW
K = 80
CPS = 25
NSUPER = EW // (K * CPS)
ROWS_T = N // NS
SW = S // NW
LANES = 16

ROW_OFFS = tuple(range(0, D, LANES)) + (DE - LANES,)


def _encode_body(src_h, dst_h, typ_h, nodep_h, relnp_h,
                 agg0_h, agg1_h,
                 agg_sh, rel_sh, sidx, didx, tidx,
                 br0, br1, br2,
                 semb0, semb1, semb2, semc0, semc1, semc2,
                 semd0, semd1, semd2):
    c = lax.axis_index("c")
    s = lax.axis_index("s")
    wid = s * NC + c
    br = [br0, br1, br2]
    semb = [semb0, semb1, semb2]
    semc = [semc0, semc1, semc2]
    semd = [semd0, semd1, semd2]
    zero = jnp.zeros((LANES,), jnp.float32)
    zeroi = jnp.zeros((LANES,), jnp.int32)

    def zb_body(j, carry):
        for off in ROW_OFFS:
            br0[j, pl.ds(off, LANES)] = zero
        return carry

    lax.fori_loop(0, K, zb_body, None)

    def didx_body(j, carry):
        for i in range(K // LANES):
            didx[j, pl.ds(i * LANES, LANES)] = zeroi
        return carry

    lax.fori_loop(0, CPS, didx_body, None)

    for q in range(7):
        pltpu.sync_copy(br0, agg_sh.at[pl.ds(s * ROWS_T + q * K, K)])
    pltpu.sync_copy(br0.at[pl.ds(0, 65)],
                    agg_sh.at[pl.ds(s * ROWS_T + 7 * K, 65)])

    @pl.when(s == 0)
    def _():
        pltpu.sync_copy(relnp_h, rel_sh)
    plsc.subcore_barrier()

    def block(g, carry):
        rbase = wid * (EW // K) + g * CPS
        pltpu.sync_copy(src_h.at[pl.ds(rbase, CPS)], sidx)
        pltpu.sync_copy(typ_h.at[pl.ds(rbase, CPS)], tidx)
        pltpu.sync_copy(dst_h.at[pl.ds(rbase, CPS)], didx)
        cpn = [None, None, None]
        cpc = [None, None, None]
        cpn[0] = pltpu.async_copy(nodep_h.at[sidx.at[0]], br[0], semb[0])
        for q in range(CPS):
            p = q % 3
            if q >= 1:
                pl_ = (q - 1) % 3
                cpc[pl_].wait()
                pltpu.async_copy(br[pl_], agg_sh.at[didx.at[q - 1]],
                                 semd[pl_], add=True)
            if q >= 2:
                pp = (q - 2) % 3
                pltpu.make_async_copy(
                    br0, agg_sh.at[didx.at[0]], semd[pp]).wait()
            if q + 1 < CPS:
                pn = (q + 1) % 3
                cpn[pn] = pltpu.async_copy(nodep_h.at[sidx.at[q + 1]],
                                           br[pn], semb[pn])
            cpn[p].wait()
            cpc[p] = pltpu.async_copy(rel_sh.at[tidx.at[q]], br[p],
                                      semc[p], add=True)
        pl_ = (CPS - 1) % 3
        cpc[pl_].wait()
        pltpu.async_copy(br[pl_], agg_sh.at[didx.at[CPS - 1]],
                         semd[pl_], add=True)
        for pp in ((CPS - 2) % 3, (CPS - 1) % 3):
            pltpu.make_async_copy(br0, agg_sh.at[didx.at[0]], semd[pp]).wait()
        return carry

    lax.fori_loop(0, NSUPER, block, None)
    plsc.subcore_barrier()

    @pl.when(c == 0)
    def _():
        pltpu.sync_copy(agg_sh.at[pl.ds(s * ROWS_T, ROWS_T)],
                        agg0_h.at[pl.ds(s * ROWS_T, ROWS_T)])

    @pl.when(c == 1)
    def _():
        pltpu.sync_copy(agg_sh.at[pl.ds(s * ROWS_T, ROWS_T)],
                        agg1_h.at[pl.ds(s * ROWS_T, ROWS_T)])


def _extract_body(sub_h, a0_h, a1_h, node_h, x_h,
                  idx, g0, g1, gn, xb, sem):
    c = lax.axis_index("c")
    s = lax.axis_index("s")
    wid = s * NC + c
    base = wid * SW
    pltpu.sync_copy(sub_h.at[pl.ds(base, SW)], idx)
    cps = [pltpu.async_copy(a0_h.at[idx], g0, sem),
           pltpu.async_copy(a1_h.at[idx], g1, sem),
           pltpu.async_copy(node_h.at[idx], gn, sem)]
    for cp in cps:
        cp.wait()

    def row(j, carry):
        dpos = jnp.full((LANES,), D, jnp.int32)
        jv = jnp.full((LANES,), j, jnp.int32)
        deg = (plsc.load_gather(g0, [jv, dpos]) +
               plsc.load_gather(g1, [jv, dpos]))
        rcp = 1.0 / jnp.maximum(deg, 1.0)
        for i in range(D // LANES):
            sl = pl.ds(i * LANES, LANES)
            xb[j, sl] = (g0[j, sl] + g1[j, sl]) * rcp + gn[j, sl]
        return carry

    lax.fori_loop(0, SW, row, None)
    pltpu.sync_copy(xb, x_h.at[pl.ds(base, SW)])


def _matmul_body(x_ref, w_ref, o_ref):
    o_ref[...] = jnp.tanh(
        jnp.dot(x_ref[...], w_ref[...], preferred_element_type=jnp.float32))


@jax.jit
def kernel(edge_index, edge_type, subgraph_nodes, node_emb, rel_emb, W):
    src = edge_index[0].reshape(E // K, K)
    dst = edge_index[1].reshape(E // K, K)
    etype = edge_type.reshape(E // K, K)
    node_pad = jnp.concatenate(
        [node_emb, jnp.ones((N, DG), jnp.float32)], axis=1)
    reln_pad = jnp.concatenate(
        [-rel_emb, jnp.zeros((rel_emb.shape[0], DG), jnp.float32)], axis=1)

    mesh = plsc.VectorSubcoreMesh(core_axis_name="c", subcore_axis_name="s")
    encode = pl.kernel(
        _encode_body,
        out_type=[jax.ShapeDtypeStruct((N, DE), jnp.float32),
                  jax.ShapeDtypeStruct((N, DE), jnp.float32)],
        mesh=mesh,
        scratch_types=[
            pltpu.VMEM_SHARED((N, DE), jnp.float32),
            pltpu.VMEM_SHARED((200, DE), jnp.float32),
            pltpu.VMEM((CPS, K), jnp.int32),
            pltpu.VMEM((CPS, K), jnp.int32),
            pltpu.VMEM((CPS, K), jnp.int32),
            pltpu.VMEM((K, DE), jnp.float32),
            pltpu.VMEM((K, DE), jnp.float32),
            pltpu.VMEM((K, DE), jnp.float32),
        ] + [pltpu.SemaphoreType.DMA] * 9,
        compiler_params=pltpu.CompilerParams(use_tc_tiling_on_sc=False),
    )
    agg0, agg1 = encode(src, dst, etype, node_pad, reln_pad)

    extract = pl.kernel(
        _extract_body,
        out_type=jax.ShapeDtypeStruct((S, D), jnp.float32),
        mesh=plsc.VectorSubcoreMesh(core_axis_name="c", subcore_axis_name="s"),
        scratch_types=[
            pltpu.VMEM((SW,), jnp.int32),
            pltpu.VMEM((SW, DE), jnp.float32),
            pltpu.VMEM((SW, DE), jnp.float32),
            pltpu.VMEM((SW, D), jnp.float32),
            pltpu.VMEM((SW, D), jnp.float32),
            pltpu.SemaphoreType.DMA,
        ],
        compiler_params=pltpu.CompilerParams(use_tc_tiling_on_sc=False,
                                             needs_layout_passes=False),
    )
    x = extract(subgraph_nodes, agg0, agg1, node_emb)

    return pl.pallas_call(
        _matmul_body,
        out_shape=jax.ShapeDtypeStruct((S, D), jnp.float32),
    )(x, W)
